# baseline (speedup 1.0000x reference)
import functools

import jax
import jax.numpy as jnp
from jax import lax
from jax.experimental import pallas as pl
from jax.experimental.pallas import tpu as pltpu
from jax.experimental.pallas import tpu_sc as plsc

_K = 4
_V = 100000
_D = 64
_B = 4096
_S = 200
_J = _S // _K
_N = _B * _S
_NW = 32
_BPW = _B // _NW
_BLK = _K * _BPW


def _make_kernel():
    mesh = plsc.VectorSubcoreMesh(core_axis_name="c", subcore_axis_name="s")

    @functools.partial(
        pl.kernel,
        mesh=mesh,
        out_type=jax.ShapeDtypeStruct((_J, _B // 2, 128), jnp.float32),
        compiler_params=pltpu.CompilerParams(use_tc_tiling_on_sc=False),
        scratch_types=[
            pltpu.VMEM((2, 4, 128), jnp.int32),
            pltpu.VMEM((2, _BLK, _D), jnp.float32),
            pltpu.VMEM((2, _BPW // 2, 128), jnp.float32),
            pltpu.SemaphoreType.DMA,
            pltpu.SemaphoreType.DMA,
            pltpu.SemaphoreType.DMA,
            pltpu.SemaphoreType.DMA,
            pltpu.SemaphoreType.DMA,
            pltpu.SemaphoreType.DMA,
        ],
    )
    def body(ids_hbm, table_hbm, out_hbm, sidx_v, rows_v, out_v,
             sg0, sg1, so0, so1, si0, si1):
        wid = lax.axis_index("c") * 16 + lax.axis_index("s")
        sgs = (sg0, sg1)
        sos = (so0, so1)
        sis = (si0, si1)

        iota = lax.iota(jnp.int32, 16)
        offv = ((iota % _K) - (iota % 2)) * _V + (iota % 2)

        def idx_copy(j, buf):
            return pltpu.make_async_copy(ids_hbm.at[j, wid], sidx_v.at[buf], sis[buf])

        def fire_gathers(j, buf):
            for s in range(4):
                for c in range(8):
                    sl = pl.ds(c * 16, 16)
                    sidx_v[buf, s, sl] = sidx_v[buf, s, sl] * 2 + offv
            for s in range(4):
                pltpu.async_copy(
                    table_hbm.at[sidx_v.at[buf].at[s]],
                    rows_v.at[buf].at[pl.ds(s * 128, 128)],
                    sgs[buf],
                )

        def wait_gathers(j, buf):
            for s in range(4):
                pltpu.make_async_copy(
                    table_hbm.at[sidx_v.at[buf].at[s]],
                    rows_v.at[buf].at[pl.ds(s * 128, 128)],
                    sgs[buf],
                ).wait()

        def out_copy(j, buf):
            return pltpu.make_async_copy(
                out_v.at[buf],
                out_hbm.at[j, pl.ds(wid * (_BPW // 2), _BPW // 2)],
                sos[buf],
            )

        idx_copy(0, 0).start()
        idx_copy(0, 0).wait()
        fire_gathers(0, 0)
        idx_copy(1, 1).start()

        def outer(jj, carry):
            for bf in range(2):
                j = jj * 2 + bf

                @pl.when(j + 1 < _J)
                def _():
                    idx_copy(j + 1, 1 - bf).wait()
                    fire_gathers(j + 1, 1 - bf)

                wait_gathers(j, bf)

                @pl.when(j + 2 < _J)
                def _():
                    idx_copy(j + 2, bf).start()

                @pl.when(j >= 2)
                def _():
                    out_copy(j - 2, bf).wait()

                rv = rows_v.at[bf]
                ov = out_v.at[bf]

                @plsc.parallel_loop(0, _BPW // 2, unroll=2)
                def _(n):
                    for h in range(2):
                        r = (n * 2 + h) * 4
                        for c in range(4):
                            ov[n, pl.ds(h * 64 + c * 16, 16)] = (
                                rv[r, pl.ds(c * 16, 16)]
                                + rv[r + 1, pl.ds(c * 16, 16)]
                                + rv[r + 2, pl.ds(c * 16, 16)]
                                + rv[r + 3, pl.ds(c * 16, 16)]
                            )

                out_copy(j, bf).start()
            return carry

        lax.fori_loop(0, _J // 2, outer, 0)

        for bf in range(2):
            out_copy(_J - 2 + bf, bf).wait()

    return body


_sc_kernel = _make_kernel()

_VC = 2048
_NG = (_V + _VC - 1) // _VC


def _tc_format_body(in_ref, out_ref):
    y0 = jnp.swapaxes(in_ref[0], 0, 1)
    y1 = jnp.swapaxes(in_ref[1], 0, 1)
    out_ref[0] = jnp.concatenate([y0, y1], axis=1)


def _format_table(tables):
    t = jnp.swapaxes(tables, 1, 2)
    out = pl.pallas_call(
        _tc_format_body,
        grid=(2, _NG),
        in_specs=[pl.BlockSpec((2, _D, _VC), lambda p, g: (p, 0, g))],
        out_specs=pl.BlockSpec((1, _VC, 128), lambda p, g: (p, g, 0)),
        out_shape=jax.ShapeDtypeStruct((2, _V, 128), jnp.float32),
    )(t)
    return out.reshape(_K * _V, _D)


def _tc_out_body(in_ref, out_ref):
    x = in_ref[0]
    y = x.reshape(_BPW // 2, 2, _D).transpose(2, 0, 1)
    out_ref[:, 0] = y.reshape(_D, _BPW).reshape(8, 8, 128)


def _format_out(out_sc):
    return pl.pallas_call(
        _tc_out_body,
        grid=(_J, _NW),
        in_specs=[pl.BlockSpec((1, _BPW // 2, 128), lambda j, tc: (j, tc, 0))],
        out_specs=pl.BlockSpec((8, 1, 8, 128), lambda j, tc: (j, tc, 0, 0)),
        out_shape=jax.ShapeDtypeStruct((_J * 8, _NW, 8, 128), jnp.float32),
    )(out_sc)


@jax.jit
def kernel(input_ids, tables):
    ids_shuf = (
        input_ids.reshape(_NW, _BPW, _J, _K)
        .transpose(2, 0, 1, 3)
        .reshape(_J, _NW, 4, 128)
    )
    table_flat = _format_table(tables)
    out_sc = _sc_kernel(ids_shuf, table_flat)
    out4 = _format_out(out_sc)
    z = out4.reshape(_J, 8, _NW, 8, 128).transpose(2, 4, 0, 1, 3)
    return z.reshape(_B, _J, _D)


# speedup vs baseline: 1.9956x; 1.9956x over previous
import functools

import jax
import jax.numpy as jnp
from jax import lax
from jax.experimental import pallas as pl
from jax.experimental.pallas import tpu as pltpu
from jax.experimental.pallas import tpu_sc as plsc

_K = 4
_V = 100000
_D = 64
_B = 4096
_S = 200
_J = _S // _K
_N = _B * _S
_NW = 32
_BPW = _B // _NW
_BLK = _K * _BPW


def _make_kernel():
    mesh = plsc.VectorSubcoreMesh(core_axis_name="c", subcore_axis_name="s")

    @functools.partial(
        pl.kernel,
        mesh=mesh,
        out_type=jax.ShapeDtypeStruct((_J, _B // 2, 128), jnp.float32),
        compiler_params=pltpu.CompilerParams(use_tc_tiling_on_sc=False),
        scratch_types=[
            pltpu.VMEM((2, 4, 128), jnp.int32),
            pltpu.VMEM((2, _BLK, _D), jnp.float32),
            pltpu.VMEM((2, _BPW // 2, 128), jnp.float32),
            pltpu.SemaphoreType.DMA,
            pltpu.SemaphoreType.DMA,
            pltpu.SemaphoreType.DMA,
            pltpu.SemaphoreType.DMA,
            pltpu.SemaphoreType.DMA,
            pltpu.SemaphoreType.DMA,
        ],
    )
    def body(ids_hbm, table_hbm, out_hbm, sidx_v, rows_v, out_v,
             sg0, sg1, so0, so1, si0, si1):
        wid = lax.axis_index("c") * 16 + lax.axis_index("s")
        sgs = (sg0, sg1)
        sos = (so0, so1)
        sis = (si0, si1)

        iota = lax.iota(jnp.int32, 16)
        offv = ((iota % _K) - (iota % 2)) * _V + (iota % 2)

        def idx_copy(j, buf):
            return pltpu.make_async_copy(ids_hbm.at[j, wid], sidx_v.at[buf], sis[buf])

        def fire_gathers(j, buf):
            for s in range(4):
                for c in range(8):
                    sl = pl.ds(c * 16, 16)
                    sidx_v[buf, s, sl] = sidx_v[buf, s, sl] * 2 + offv
            for s in range(4):
                pltpu.async_copy(
                    table_hbm.at[sidx_v.at[buf].at[s]],
                    rows_v.at[buf].at[pl.ds(s * 128, 128)],
                    sgs[buf],
                )

        def wait_gathers(j, buf):
            for s in range(4):
                pltpu.make_async_copy(
                    table_hbm.at[sidx_v.at[buf].at[s]],
                    rows_v.at[buf].at[pl.ds(s * 128, 128)],
                    sgs[buf],
                ).wait()

        def out_copy(j, buf):
            return pltpu.make_async_copy(
                out_v.at[buf],
                out_hbm.at[j, pl.ds(wid * (_BPW // 2), _BPW // 2)],
                sos[buf],
            )

        idx_copy(0, 0).start()
        idx_copy(0, 0).wait()
        fire_gathers(0, 0)
        idx_copy(1, 1).start()

        def outer(jj, carry):
            for bf in range(2):
                j = jj * 2 + bf

                @pl.when(j + 1 < _J)
                def _():
                    idx_copy(j + 1, 1 - bf).wait()
                    fire_gathers(j + 1, 1 - bf)

                wait_gathers(j, bf)

                @pl.when(j + 2 < _J)
                def _():
                    idx_copy(j + 2, bf).start()

                @pl.when(j >= 2)
                def _():
                    out_copy(j - 2, bf).wait()

                rv = rows_v.at[bf]
                ov = out_v.at[bf]

                @plsc.parallel_loop(0, _BPW // 2, unroll=2)
                def _(n):
                    for h in range(2):
                        r = (n + h * 64) * 4
                        for c in range(4):
                            ov[n, pl.ds(h * 64 + c * 16, 16)] = (
                                rv[r, pl.ds(c * 16, 16)]
                                + rv[r + 1, pl.ds(c * 16, 16)]
                                + rv[r + 2, pl.ds(c * 16, 16)]
                                + rv[r + 3, pl.ds(c * 16, 16)]
                            )

                out_copy(j, bf).start()
            return carry

        lax.fori_loop(0, _J // 2, outer, 0)

        for bf in range(2):
            out_copy(_J - 2 + bf, bf).wait()

    return body


_sc_kernel = _make_kernel()

_VC = 2048
_NG = (_V + _VC - 1) // _VC


def _tc_format_body(in_ref, out_ref):
    y0 = jnp.swapaxes(in_ref[0], 0, 1)
    y1 = jnp.swapaxes(in_ref[1], 0, 1)
    out_ref[0] = jnp.concatenate([y0, y1], axis=1)


def _format_table(tables):
    t = jnp.swapaxes(tables, 1, 2)
    out = pl.pallas_call(
        _tc_format_body,
        grid=(2, _NG),
        in_specs=[pl.BlockSpec((2, _D, _VC), lambda p, g: (p, 0, g))],
        out_specs=pl.BlockSpec((1, _VC, 128), lambda p, g: (p, g, 0)),
        out_shape=jax.ShapeDtypeStruct((2, _V, 128), jnp.float32),
    )(t)
    return out.reshape(_K * _V, _D)


def _tc_out_body(in_ref, out_ref):
    x = in_ref[0]
    ya = jnp.swapaxes(x[:, :_D], 0, 1)
    yb = jnp.swapaxes(x[:, _D:], 0, 1)
    y = jnp.concatenate([ya, yb], axis=1)
    out_ref[:, 0] = y.reshape(8, 8, 128)


def _format_out(out_sc):
    return pl.pallas_call(
        _tc_out_body,
        grid=(_J, _NW),
        in_specs=[pl.BlockSpec((1, _BPW // 2, 128), lambda j, tc: (j, tc, 0))],
        out_specs=pl.BlockSpec((8, 1, 8, 128), lambda j, tc: (j, tc, 0, 0)),
        out_shape=jax.ShapeDtypeStruct((_J * 8, _NW, 8, 128), jnp.float32),
    )(out_sc)


@jax.jit
def kernel(input_ids, tables):
    ids_shuf = (
        input_ids.reshape(_NW, _BPW, _J, _K)
        .transpose(2, 0, 1, 3)
        .reshape(_J, _NW, 4, 128)
    )
    table_flat = _format_table(tables)
    out_sc = _sc_kernel(ids_shuf, table_flat)
    out4 = _format_out(out_sc)
    z = out4.reshape(_J, 8, _NW, 8, 128).transpose(2, 4, 0, 1, 3)
    return z.reshape(_B, _J, _D)


# speedup vs baseline: 6.8032x; 3.4091x over previous
import functools

import jax
import jax.numpy as jnp
import numpy as np
from jax import lax
from jax.experimental import pallas as pl
from jax.experimental.pallas import tpu as pltpu
from jax.experimental.pallas import tpu_sc as plsc

_K = 4
_V = 100000
_D = 64
_B = 4096
_S = 200
_J = _S // _K
_N = _B * _S
_NW = 32
_BPW = _B // _NW
_BLK = _K * _BPW


def _make_kernel():
    mesh = plsc.VectorSubcoreMesh(core_axis_name="c", subcore_axis_name="s")

    @functools.partial(
        pl.kernel,
        mesh=mesh,
        out_type=jax.ShapeDtypeStruct((_J, _B // 2, 128), jnp.float32),
        compiler_params=pltpu.CompilerParams(use_tc_tiling_on_sc=False),
        scratch_types=[
            pltpu.VMEM((2, 4, 128), jnp.int32),
            pltpu.VMEM((2, _BLK, _D), jnp.float32),
            pltpu.VMEM((2, _BPW // 2, 128), jnp.float32),
            pltpu.SemaphoreType.DMA,
            pltpu.SemaphoreType.DMA,
            pltpu.SemaphoreType.DMA,
            pltpu.SemaphoreType.DMA,
            pltpu.SemaphoreType.DMA,
            pltpu.SemaphoreType.DMA,
        ],
    )
    def body(ids_hbm, table_hbm, out_hbm, sidx_v, rows_v, out_v,
             sg0, sg1, so0, so1, si0, si1):
        wid = lax.axis_index("c") * 16 + lax.axis_index("s")
        sgs = (sg0, sg1)
        sos = (so0, so1)
        sis = (si0, si1)

        iota = lax.iota(jnp.int32, 16)
        offv = ((iota % _K) - (iota % 2)) * _V + (iota % 2)

        def idx_copy(j, buf):
            return pltpu.make_async_copy(ids_hbm.at[j, wid], sidx_v.at[buf], sis[buf])

        def fire_gathers(j, buf):
            for s in range(4):
                for c in range(8):
                    sl = pl.ds(c * 16, 16)
                    sidx_v[buf, s, sl] = sidx_v[buf, s, sl] * 2 + offv
            for s in range(4):
                pltpu.async_copy(
                    table_hbm.at[sidx_v.at[buf].at[s]],
                    rows_v.at[buf].at[pl.ds(s * 128, 128)],
                    sgs[buf],
                )

        def wait_gathers(j, buf):
            for s in range(4):
                pltpu.make_async_copy(
                    table_hbm.at[sidx_v.at[buf].at[s]],
                    rows_v.at[buf].at[pl.ds(s * 128, 128)],
                    sgs[buf],
                ).wait()

        def out_copy(j, buf):
            return pltpu.make_async_copy(
                out_v.at[buf],
                out_hbm.at[j, pl.ds(wid * (_BPW // 2), _BPW // 2)],
                sos[buf],
            )

        idx_copy(0, 0).start()
        idx_copy(0, 0).wait()
        fire_gathers(0, 0)
        idx_copy(1, 1).start()

        def outer(jj, carry):
            for bf in range(2):
                j = jj * 2 + bf

                @pl.when(j + 1 < _J)
                def _():
                    idx_copy(j + 1, 1 - bf).wait()
                    fire_gathers(j + 1, 1 - bf)

                wait_gathers(j, bf)

                @pl.when(j + 2 < _J)
                def _():
                    idx_copy(j + 2, bf).start()

                @pl.when(j >= 2)
                def _():
                    out_copy(j - 2, bf).wait()

                rv = rows_v.at[bf]
                ov = out_v.at[bf]

                @plsc.parallel_loop(0, _BPW // 2, unroll=2)
                def _(n):
                    for h in range(2):
                        r = (n * 2 + h) * 4
                        for c in range(4):
                            ov[n, pl.ds(h * 64 + c * 16, 16)] = (
                                rv[r, pl.ds(c * 16, 16)]
                                + rv[r + 1, pl.ds(c * 16, 16)]
                                + rv[r + 2, pl.ds(c * 16, 16)]
                                + rv[r + 3, pl.ds(c * 16, 16)]
                            )

                out_copy(j, bf).start()
            return carry

        lax.fori_loop(0, _J // 2, outer, 0)

        for bf in range(2):
            out_copy(_J - 2 + bf, bf).wait()

    return body


_sc_kernel = _make_kernel()

_VC = 2048
_NG = (_V + _VC - 1) // _VC


def _tc_format_body(in_ref, out_ref):
    y0 = jnp.swapaxes(in_ref[0], 0, 1)
    y1 = jnp.swapaxes(in_ref[1], 0, 1)
    out_ref[0] = jnp.concatenate([y0, y1], axis=1)


def _format_table(tables):
    t = jnp.swapaxes(tables, 1, 2)
    out = pl.pallas_call(
        _tc_format_body,
        grid=(2, _NG),
        in_specs=[pl.BlockSpec((2, _D, _VC), lambda p, g: (p, 0, g))],
        out_specs=pl.BlockSpec((1, _VC, 128), lambda p, g: (p, g, 0)),
        out_shape=jax.ShapeDtypeStruct((2, _V, 128), jnp.float32),
    )(t)
    return out.reshape(_K * _V, _D)


def _tc_out_body(in_ref, out_ref):
    x = in_ref[0]
    ya = jnp.swapaxes(x[:, :_D], 0, 1)
    yb = jnp.swapaxes(x[:, _D:], 0, 1)
    out_ref[...] = jnp.concatenate([ya, yb], axis=1)


def _format_out(out_sc):
    return pl.pallas_call(
        _tc_out_body,
        grid=(_J,),
        in_specs=[pl.BlockSpec((1, _B // 2, 128), lambda j: (j, 0, 0))],
        out_specs=pl.BlockSpec((_D, _B), lambda j: (j, 0)),
        out_shape=jax.ShapeDtypeStruct((_J * _D, _B), jnp.float32),
    )(out_sc)


_w = np.arange(_NW)[:, None]
_n = np.arange(_BPW // 2)[None, :]
_PERM = np.stack([64 * _w + _n, 2048 + 64 * _w + _n], axis=2).reshape(-1)


@jax.jit
def kernel(input_ids, tables):
    ids_shuf = (
        input_ids[jnp.asarray(_PERM)]
        .reshape(_NW, _BPW, _J, _K)
        .transpose(2, 0, 1, 3)
        .reshape(_J, _NW, 4, 128)
    )
    table_flat = _format_table(tables)
    out_sc = _sc_kernel(ids_shuf, table_flat)
    out2d = _format_out(out_sc)
    return out2d.reshape(_J, _D, _B).transpose(2, 0, 1)


# speedup vs baseline: 7.3661x; 1.0827x over previous
import functools

import jax
import jax.numpy as jnp
import numpy as np
from jax import lax
from jax.experimental import pallas as pl
from jax.experimental.pallas import tpu as pltpu
from jax.experimental.pallas import tpu_sc as plsc

_K = 4
_V = 100000
_D = 64
_B = 4096
_S = 200
_J = _S // _K
_N = _B * _S
_NW = 32
_BPW = _B // _NW
_BLK = _K * _BPW


def _make_kernel():
    mesh = plsc.VectorSubcoreMesh(core_axis_name="c", subcore_axis_name="s")

    @functools.partial(
        pl.kernel,
        mesh=mesh,
        out_type=jax.ShapeDtypeStruct((_J, _B // 2, 128), jnp.float32),
        compiler_params=pltpu.CompilerParams(use_tc_tiling_on_sc=False),
        scratch_types=[
            pltpu.VMEM((2, 4, 128), jnp.int32),
            pltpu.VMEM((2, _BLK, _D), jnp.float32),
            pltpu.VMEM((2, _BPW // 2, 128), jnp.float32),
            pltpu.SemaphoreType.DMA,
            pltpu.SemaphoreType.DMA,
            pltpu.SemaphoreType.DMA,
            pltpu.SemaphoreType.DMA,
            pltpu.SemaphoreType.DMA,
            pltpu.SemaphoreType.DMA,
        ],
    )
    def body(ids_hbm, table_hbm, out_hbm, sidx_v, rows_v, out_v,
             sg0, sg1, so0, so1, si0, si1):
        wid = lax.axis_index("c") * 16 + lax.axis_index("s")
        sgs = (sg0, sg1)
        sos = (so0, so1)
        sis = (si0, si1)

        iota = lax.iota(jnp.int32, 16)
        offv = ((iota % _K) - (iota % 2)) * _V + (iota % 2)

        def idx_copy(j, buf):
            return pltpu.make_async_copy(ids_hbm.at[j, wid], sidx_v.at[buf], sis[buf])

        def fire_gathers(j, buf):
            for s in range(4):
                for c in range(8):
                    sl = pl.ds(c * 16, 16)
                    sidx_v[buf, s, sl] = sidx_v[buf, s, sl] * 2 + offv
            for s in range(4):
                pltpu.async_copy(
                    table_hbm.at[sidx_v.at[buf].at[s]],
                    rows_v.at[buf].at[pl.ds(s * 128, 128)],
                    sgs[buf],
                )

        def wait_gathers(j, buf):
            for s in range(4):
                pltpu.make_async_copy(
                    table_hbm.at[sidx_v.at[buf].at[s]],
                    rows_v.at[buf].at[pl.ds(s * 128, 128)],
                    sgs[buf],
                ).wait()

        def out_copy(j, buf):
            return pltpu.make_async_copy(
                out_v.at[buf],
                out_hbm.at[j, pl.ds(wid * (_BPW // 2), _BPW // 2)],
                sos[buf],
            )

        idx_copy(0, 0).start()
        idx_copy(0, 0).wait()
        fire_gathers(0, 0)
        idx_copy(1, 1).start()

        def outer(jj, carry):
            for bf in range(2):
                j = jj * 2 + bf

                @pl.when(j + 1 < _J)
                def _():
                    idx_copy(j + 1, 1 - bf).wait()
                    fire_gathers(j + 1, 1 - bf)

                wait_gathers(j, bf)

                @pl.when(j + 2 < _J)
                def _():
                    idx_copy(j + 2, bf).start()

                @pl.when(j >= 2)
                def _():
                    out_copy(j - 2, bf).wait()

                rv = rows_v.at[bf]
                ov = out_v.at[bf]

                @plsc.parallel_loop(0, _BPW // 2, unroll=4)
                def _(n):
                    for h in range(2):
                        r = (n * 2 + h) * 4
                        for c in range(4):
                            ov[n, pl.ds(h * 64 + c * 16, 16)] = (
                                rv[r, pl.ds(c * 16, 16)]
                                + rv[r + 1, pl.ds(c * 16, 16)]
                                + rv[r + 2, pl.ds(c * 16, 16)]
                                + rv[r + 3, pl.ds(c * 16, 16)]
                            )

                out_copy(j, bf).start()
            return carry

        lax.fori_loop(0, _J // 2, outer, 0)

        for bf in range(2):
            out_copy(_J - 2 + bf, bf).wait()

    return body


_sc_kernel = _make_kernel()

_VC = 4096
_NG = (_V + _VC - 1) // _VC


def _tc_format_body(in_ref, out_ref):
    y0 = jnp.swapaxes(in_ref[0], 0, 1)
    y1 = jnp.swapaxes(in_ref[1], 0, 1)
    out_ref[0] = jnp.concatenate([y0, y1], axis=1)


def _format_table(tables):
    t = jnp.swapaxes(tables, 1, 2)
    out = pl.pallas_call(
        _tc_format_body,
        grid=(2, _NG),
        in_specs=[pl.BlockSpec((2, _D, _VC), lambda p, g: (p, 0, g))],
        out_specs=pl.BlockSpec((1, _VC, 128), lambda p, g: (p, g, 0)),
        out_shape=jax.ShapeDtypeStruct((2, _V, 128), jnp.float32),
    )(t)
    return out.reshape(_K * _V, _D)


def _tc_out_body(in_ref, out_ref):
    x = in_ref[0]
    ya = jnp.swapaxes(x[:, :_D], 0, 1)
    yb = jnp.swapaxes(x[:, _D:], 0, 1)
    out_ref[...] = jnp.concatenate([ya, yb], axis=1)


def _format_out(out_sc):
    return pl.pallas_call(
        _tc_out_body,
        grid=(_J,),
        in_specs=[pl.BlockSpec((1, _B // 2, 128), lambda j: (j, 0, 0))],
        out_specs=pl.BlockSpec((_D, _B), lambda j: (j, 0)),
        out_shape=jax.ShapeDtypeStruct((_J * _D, _B), jnp.float32),
    )(out_sc)


_w = np.arange(_NW)[:, None]
_n = np.arange(_BPW // 2)[None, :]
_PERM = np.stack([64 * _w + _n, 2048 + 64 * _w + _n], axis=2).reshape(-1)


@jax.jit
def kernel(input_ids, tables):
    ids_shuf = (
        input_ids[jnp.asarray(_PERM)]
        .reshape(_NW, _BPW, _J, _K)
        .transpose(2, 0, 1, 3)
        .reshape(_J, _NW, 4, 128)
    )
    table_flat = _format_table(tables)
    out_sc = _sc_kernel(ids_shuf, table_flat)
    out2d = _format_out(out_sc)
    return out2d.reshape(_J, _D, _B).transpose(2, 0, 1)


# speedup vs baseline: 7.9154x; 1.0746x over previous
import functools

import jax
import jax.numpy as jnp
import numpy as np
from jax import lax
from jax.experimental import pallas as pl
from jax.experimental.pallas import tpu as pltpu
from jax.experimental.pallas import tpu_sc as plsc

_K = 4
_V = 100000
_D = 64
_B = 4096
_S = 200
_J = _S // _K
_N = _B * _S
_NW = 32
_BPW = _B // _NW
_BLK = _K * _BPW


def _make_kernel():
    mesh = plsc.VectorSubcoreMesh(core_axis_name="c", subcore_axis_name="s")

    @functools.partial(
        pl.kernel,
        mesh=mesh,
        out_type=jax.ShapeDtypeStruct((_J, _B // 2, 128), jnp.float32),
        compiler_params=pltpu.CompilerParams(use_tc_tiling_on_sc=False),
        scratch_types=[
            pltpu.VMEM((2, 4, 128), jnp.int32),
            pltpu.VMEM((2, _BLK, _D), jnp.float32),
            pltpu.VMEM((2, _BPW // 2, 128), jnp.float32),
            pltpu.SemaphoreType.DMA,
            pltpu.SemaphoreType.DMA,
            pltpu.SemaphoreType.DMA,
            pltpu.SemaphoreType.DMA,
            pltpu.SemaphoreType.DMA,
            pltpu.SemaphoreType.DMA,
        ],
    )
    def body(ids_hbm, table_hbm, out_hbm, sidx_v, rows_v, out_v,
             sg0, sg1, so0, so1, si0, si1):
        wid = lax.axis_index("c") * 16 + lax.axis_index("s")
        sgs = (sg0, sg1)
        sos = (so0, so1)
        sis = (si0, si1)

        iota = lax.iota(jnp.int32, 16)
        offv = ((iota % _K) - (iota % 2)) * _V + (iota % 2)

        def idx_copy(j, buf):
            return pltpu.make_async_copy(ids_hbm.at[j, wid], sidx_v.at[buf], sis[buf])

        def fire_gathers(j, buf):
            for s in range(4):
                for c in range(8):
                    sl = pl.ds(c * 16, 16)
                    sidx_v[buf, s, sl] = sidx_v[buf, s, sl] * 2 + offv
            for s in range(4):
                pltpu.async_copy(
                    table_hbm.at[sidx_v.at[buf].at[s]],
                    rows_v.at[buf].at[pl.ds(s * 128, 128)],
                    sgs[buf],
                )

        def wait_gathers(j, buf):
            for s in range(4):
                pltpu.make_async_copy(
                    table_hbm.at[sidx_v.at[buf].at[s]],
                    rows_v.at[buf].at[pl.ds(s * 128, 128)],
                    sgs[buf],
                ).wait()

        def out_copy(j, buf):
            return pltpu.make_async_copy(
                out_v.at[buf],
                out_hbm.at[j, pl.ds(wid * (_BPW // 2), _BPW // 2)],
                sos[buf],
            )

        idx_copy(0, 0).start()
        idx_copy(0, 0).wait()
        fire_gathers(0, 0)
        idx_copy(1, 1).start()

        def outer(jj, carry):
            for bf in range(2):
                j = jj * 2 + bf

                @pl.when(j + 1 < _J)
                def _():
                    idx_copy(j + 1, 1 - bf).wait()
                    fire_gathers(j + 1, 1 - bf)

                wait_gathers(j, bf)

                @pl.when(j + 2 < _J)
                def _():
                    idx_copy(j + 2, bf).start()

                @pl.when(j >= 2)
                def _():
                    out_copy(j - 2, bf).wait()

                rv = rows_v.at[bf]
                ov = out_v.at[bf]

                @plsc.parallel_loop(0, _BPW // 2, unroll=4)
                def _(n):
                    for h in range(2):
                        r = (n * 2 + h) * 4
                        for c in range(4):
                            ov[n, pl.ds(h * 64 + c * 16, 16)] = (
                                rv[r, pl.ds(c * 16, 16)]
                                + rv[r + 1, pl.ds(c * 16, 16)]
                                + rv[r + 2, pl.ds(c * 16, 16)]
                                + rv[r + 3, pl.ds(c * 16, 16)]
                            )

                out_copy(j, bf).start()
            return carry

        lax.fori_loop(0, _J // 2, outer, 0)

        for bf in range(2):
            out_copy(_J - 2 + bf, bf).wait()

    return body


_sc_kernel = _make_kernel()

_VC = 8192
_NG = (_V + _VC - 1) // _VC


def _tc_format_body(in_ref, out_ref):
    y0 = jnp.swapaxes(in_ref[0], 0, 1)
    y1 = jnp.swapaxes(in_ref[1], 0, 1)
    out_ref[0] = jnp.concatenate([y0, y1], axis=1)


def _format_table(tables):
    t = jnp.swapaxes(tables, 1, 2)
    out = pl.pallas_call(
        _tc_format_body,
        grid=(2, _NG),
        in_specs=[pl.BlockSpec((2, _D, _VC), lambda p, g: (p, 0, g))],
        out_specs=pl.BlockSpec((1, _VC, 128), lambda p, g: (p, g, 0)),
        out_shape=jax.ShapeDtypeStruct((2, _V, 128), jnp.float32),
    )(t)
    return out.reshape(_K * _V, _D)


def _tc_out_body(in_ref, out_ref):
    x = in_ref[0]
    ya = jnp.swapaxes(x[:, :_D], 0, 1)
    yb = jnp.swapaxes(x[:, _D:], 0, 1)
    out_ref[...] = jnp.concatenate([ya, yb], axis=1)


def _format_out(out_sc):
    return pl.pallas_call(
        _tc_out_body,
        grid=(_J,),
        in_specs=[pl.BlockSpec((1, _B // 2, 128), lambda j: (j, 0, 0))],
        out_specs=pl.BlockSpec((_D, _B), lambda j: (j, 0)),
        out_shape=jax.ShapeDtypeStruct((_J * _D, _B), jnp.float32),
    )(out_sc)


_w = np.arange(_NW)[:, None]
_n = np.arange(_BPW // 2)[None, :]
_PERM = np.stack([64 * _w + _n, 2048 + 64 * _w + _n], axis=2).reshape(-1)


_pj, _pw, _ps = np.meshgrid(np.arange(_J), np.arange(_NW), np.arange(_BLK), indexing="ij")
_P_FULL = _PERM.reshape(_NW, _BPW)[_pw, _ps // 4] * _S + 4 * _pj + (_ps % 4)
_P_FULL = _P_FULL.reshape(-1).astype(np.int32)


@jax.jit
def kernel(input_ids, tables):
    ids_shuf = (
        input_ids.reshape(_N)[jnp.asarray(_P_FULL)]
        .reshape(_J, _NW, 4, 128)
    )
    table_flat = _format_table(tables)
    out_sc = _sc_kernel(ids_shuf, table_flat)
    out2d = _format_out(out_sc)
    return out2d.reshape(_J, _D, _B).transpose(2, 0, 1)


# speedup vs baseline: 8.2373x; 1.0407x over previous
import functools

import jax
import jax.numpy as jnp
import numpy as np
from jax import lax
from jax.experimental import pallas as pl
from jax.experimental.pallas import tpu as pltpu
from jax.experimental.pallas import tpu_sc as plsc

_K = 4
_V = 100000
_D = 64
_B = 4096
_S = 200
_J = _S // _K
_N = _B * _S
_NW = 32
_BPW = _B // _NW
_BLK = _K * _BPW


def _make_kernel():
    mesh = plsc.VectorSubcoreMesh(core_axis_name="c", subcore_axis_name="s")

    @functools.partial(
        pl.kernel,
        mesh=mesh,
        out_type=jax.ShapeDtypeStruct((_J, _B // 2, 128), jnp.float32),
        compiler_params=pltpu.CompilerParams(use_tc_tiling_on_sc=False),
        scratch_types=[
            pltpu.VMEM((2, 4, 128), jnp.int32),
            pltpu.VMEM((2, _BLK, _D), jnp.float32),
            pltpu.VMEM((2, _BPW // 2, 128), jnp.float32),
            pltpu.SemaphoreType.DMA,
            pltpu.SemaphoreType.DMA,
            pltpu.SemaphoreType.DMA,
            pltpu.SemaphoreType.DMA,
            pltpu.SemaphoreType.DMA,
            pltpu.SemaphoreType.DMA,
        ],
    )
    def body(ids_hbm, table_hbm, out_hbm, sidx_v, rows_v, out_v,
             sg0, sg1, so0, so1, si0, si1):
        wid = lax.axis_index("c") * 16 + lax.axis_index("s")
        sgs = (sg0, sg1)
        sos = (so0, so1)
        sis = (si0, si1)

        iota = lax.iota(jnp.int32, 16)
        offv = ((iota % _K) - (iota % 2)) * _V + (iota % 2)

        def idx_copy(j, buf):
            return pltpu.make_async_copy(ids_hbm.at[j, wid], sidx_v.at[buf], sis[buf])

        def fire_gathers(j, buf):
            for s in range(4):
                for c in range(8):
                    sl = pl.ds(c * 16, 16)
                    sidx_v[buf, s, sl] = sidx_v[buf, s, sl] * 2 + offv
            for s in range(4):
                pltpu.async_copy(
                    table_hbm.at[sidx_v.at[buf].at[s]],
                    rows_v.at[buf].at[pl.ds(s * 128, 128)],
                    sgs[buf],
                )

        def wait_gathers(j, buf):
            for s in range(4):
                pltpu.make_async_copy(
                    table_hbm.at[sidx_v.at[buf].at[s]],
                    rows_v.at[buf].at[pl.ds(s * 128, 128)],
                    sgs[buf],
                ).wait()

        def out_copy(j, buf):
            return pltpu.make_async_copy(
                out_v.at[buf],
                out_hbm.at[j, pl.ds(wid * (_BPW // 2), _BPW // 2)],
                sos[buf],
            )

        idx_copy(0, 0).start()
        idx_copy(0, 0).wait()
        fire_gathers(0, 0)
        idx_copy(1, 1).start()

        def outer(jj, carry):
            for bf in range(2):
                j = jj * 2 + bf

                @pl.when(j + 1 < _J)
                def _():
                    idx_copy(j + 1, 1 - bf).wait()
                    fire_gathers(j + 1, 1 - bf)

                wait_gathers(j, bf)

                @pl.when(j + 2 < _J)
                def _():
                    idx_copy(j + 2, bf).start()

                @pl.when(j >= 2)
                def _():
                    out_copy(j - 2, bf).wait()

                rv = rows_v.at[bf]
                ov = out_v.at[bf]

                @plsc.parallel_loop(0, _BPW // 2, unroll=4)
                def _(n):
                    for h in range(2):
                        r = (n * 2 + h) * 4
                        for c in range(4):
                            ov[n, pl.ds(h * 64 + c * 16, 16)] = (
                                rv[r, pl.ds(c * 16, 16)]
                                + rv[r + 1, pl.ds(c * 16, 16)]
                                + rv[r + 2, pl.ds(c * 16, 16)]
                                + rv[r + 3, pl.ds(c * 16, 16)]
                            )

                out_copy(j, bf).start()
            return carry

        lax.fori_loop(0, _J // 2, outer, 0)

        for bf in range(2):
            out_copy(_J - 2 + bf, bf).wait()

    return body


_sc_kernel = _make_kernel()

_VC = 16384
_NG = (_V + _VC - 1) // _VC


def _tc_format_body(in_ref, out_ref):
    y0 = jnp.swapaxes(in_ref[0], 0, 1)
    y1 = jnp.swapaxes(in_ref[1], 0, 1)
    out_ref[0] = jnp.concatenate([y0, y1], axis=1)


def _format_table(tables):
    t = jnp.swapaxes(tables, 1, 2)
    out = pl.pallas_call(
        _tc_format_body,
        grid=(2, _NG),
        in_specs=[pl.BlockSpec((2, _D, _VC), lambda p, g: (p, 0, g))],
        out_specs=pl.BlockSpec((1, _VC, 128), lambda p, g: (p, g, 0)),
        out_shape=jax.ShapeDtypeStruct((2, _V, 128), jnp.float32),
    )(t)
    return out.reshape(_K * _V, _D)


def _tc_out_body(in_ref, out_ref):
    for q in range(2):
        x = in_ref[q]
        ya = jnp.swapaxes(x[:, :_D], 0, 1)
        yb = jnp.swapaxes(x[:, _D:], 0, 1)
        out_ref[pl.ds(q * _D, _D), :] = jnp.concatenate([ya, yb], axis=1)


def _format_out(out_sc):
    return pl.pallas_call(
        _tc_out_body,
        grid=(_J // 2,),
        in_specs=[pl.BlockSpec((2, _B // 2, 128), lambda j: (j, 0, 0))],
        out_specs=pl.BlockSpec((2 * _D, _B), lambda j: (j, 0)),
        out_shape=jax.ShapeDtypeStruct((_J * _D, _B), jnp.float32),
    )(out_sc)


_w = np.arange(_NW)[:, None]
_n = np.arange(_BPW // 2)[None, :]
_PERM = np.stack([64 * _w + _n, 2048 + 64 * _w + _n], axis=2).reshape(-1)


_pj, _pw, _ps = np.meshgrid(np.arange(_J), np.arange(_NW), np.arange(_BLK), indexing="ij")
_P_FULL = _PERM.reshape(_NW, _BPW)[_pw, _ps // 4] * _S + 4 * _pj + (_ps % 4)
_P_FULL = _P_FULL.reshape(-1).astype(np.int32)


@jax.jit
def kernel(input_ids, tables):
    ids_shuf = (
        input_ids.reshape(_N)[jnp.asarray(_P_FULL)]
        .reshape(_J, _NW, 4, 128)
    )
    table_flat = _format_table(tables)
    out_sc = _sc_kernel(ids_shuf, table_flat)
    out2d = _format_out(out_sc)
    return out2d.reshape(_J, _D, _B).transpose(2, 0, 1)


# speedup vs baseline: 8.4559x; 1.0265x over previous
import functools

import jax
import jax.numpy as jnp
import numpy as np
from jax import lax
from jax.experimental import pallas as pl
from jax.experimental.pallas import tpu as pltpu
from jax.experimental.pallas import tpu_sc as plsc

_K = 4
_V = 100000
_D = 64
_B = 4096
_S = 200
_J = _S // _K
_N = _B * _S
_NW = 32
_BPW = _B // _NW
_BLK = _K * _BPW


def _make_kernel():
    mesh = plsc.VectorSubcoreMesh(core_axis_name="c", subcore_axis_name="s")

    @functools.partial(
        pl.kernel,
        mesh=mesh,
        out_type=jax.ShapeDtypeStruct((_J, _B // 2, 128), jnp.float32),
        compiler_params=pltpu.CompilerParams(use_tc_tiling_on_sc=False),
        scratch_types=[
            pltpu.VMEM((2, 4, 128), jnp.int32),
            pltpu.VMEM((2, _BLK, _D), jnp.float32),
            pltpu.VMEM((2, _BPW // 2, 128), jnp.float32),
            pltpu.SemaphoreType.DMA,
            pltpu.SemaphoreType.DMA,
            pltpu.SemaphoreType.DMA,
            pltpu.SemaphoreType.DMA,
            pltpu.SemaphoreType.DMA,
            pltpu.SemaphoreType.DMA,
        ],
    )
    def body(ids_hbm, table_hbm, out_hbm, sidx_v, rows_v, out_v,
             sg0, sg1, so0, so1, si0, si1):
        wid = lax.axis_index("c") * 16 + lax.axis_index("s")
        sgs = (sg0, sg1)
        sos = (so0, so1)
        sis = (si0, si1)

        iota = lax.iota(jnp.int32, 16)
        offv = ((iota % _K) - (iota % 2)) * _V + (iota % 2)

        def idx_copy(j, buf):
            return pltpu.make_async_copy(ids_hbm.at[j, wid], sidx_v.at[buf], sis[buf])

        def fire_gathers(j, buf):
            for s in range(4):
                for c in range(8):
                    sl = pl.ds(c * 16, 16)
                    sidx_v[buf, s, sl] = sidx_v[buf, s, sl] * 2 + offv
            for s in range(4):
                pltpu.async_copy(
                    table_hbm.at[sidx_v.at[buf].at[s]],
                    rows_v.at[buf].at[pl.ds(s * 128, 128)],
                    sgs[buf],
                )

        def wait_gathers(j, buf):
            for s in range(4):
                pltpu.make_async_copy(
                    table_hbm.at[sidx_v.at[buf].at[s]],
                    rows_v.at[buf].at[pl.ds(s * 128, 128)],
                    sgs[buf],
                ).wait()

        def out_copy(j, buf):
            return pltpu.make_async_copy(
                out_v.at[buf],
                out_hbm.at[j, pl.ds(wid * (_BPW // 2), _BPW // 2)],
                sos[buf],
            )

        idx_copy(0, 0).start()
        idx_copy(0, 0).wait()
        fire_gathers(0, 0)
        idx_copy(1, 1).start()

        def outer(jj, carry):
            for bf in range(2):
                j = jj * 2 + bf

                @pl.when(j + 1 < _J)
                def _():
                    idx_copy(j + 1, 1 - bf).wait()
                    fire_gathers(j + 1, 1 - bf)

                wait_gathers(j, bf)

                @pl.when(j + 2 < _J)
                def _():
                    idx_copy(j + 2, bf).start()

                @pl.when(j >= 2)
                def _():
                    out_copy(j - 2, bf).wait()

                rv = rows_v.at[bf]
                ov = out_v.at[bf]

                @plsc.parallel_loop(0, _BPW // 2, unroll=4)
                def _(n):
                    for h in range(2):
                        r = (n * 2 + h) * 4
                        for c in range(4):
                            ov[n, pl.ds(h * 64 + c * 16, 16)] = (
                                rv[r, pl.ds(c * 16, 16)]
                                + rv[r + 1, pl.ds(c * 16, 16)]
                                + rv[r + 2, pl.ds(c * 16, 16)]
                                + rv[r + 3, pl.ds(c * 16, 16)]
                            )

                out_copy(j, bf).start()
            return carry

        lax.fori_loop(0, _J // 2, outer, 0)

        for bf in range(2):
            out_copy(_J - 2 + bf, bf).wait()

    return body


_sc_kernel = _make_kernel()

_VC = 16384
_NG = (_V + _VC - 1) // _VC


def _tc_format_body(in_ref, out_ref):
    y0 = jnp.swapaxes(in_ref[0], 0, 1)
    y1 = jnp.swapaxes(in_ref[1], 0, 1)
    out_ref[0] = jnp.concatenate([y0, y1], axis=1)


def _format_table(tables):
    t = jnp.swapaxes(tables, 1, 2)
    out = pl.pallas_call(
        _tc_format_body,
        grid=(2, _NG),
        in_specs=[pl.BlockSpec((2, _D, _VC), lambda p, g: (p, 0, g))],
        out_specs=pl.BlockSpec((1, _VC, 128), lambda p, g: (p, g, 0)),
        out_shape=jax.ShapeDtypeStruct((2, _V, 128), jnp.float32),
    )(t)
    return out.reshape(_K * _V, _D)


def _tc_out_body(in_ref, out_ref):
    for q in range(5):
        x = in_ref[q]
        ya = jnp.swapaxes(x[:, :_D], 0, 1)
        yb = jnp.swapaxes(x[:, _D:], 0, 1)
        out_ref[pl.ds(q * _D, _D), :] = jnp.concatenate([ya, yb], axis=1)


def _format_out(out_sc):
    return pl.pallas_call(
        _tc_out_body,
        grid=(_J // 5,),
        in_specs=[pl.BlockSpec((5, _B // 2, 128), lambda j: (j, 0, 0))],
        out_specs=pl.BlockSpec((5 * _D, _B), lambda j: (j, 0)),
        out_shape=jax.ShapeDtypeStruct((_J * _D, _B), jnp.float32),
    )(out_sc)


_w = np.arange(_NW)[:, None]
_n = np.arange(_BPW // 2)[None, :]
_PERM = np.stack([64 * _w + _n, 2048 + 64 * _w + _n], axis=2).reshape(-1)


_pj, _pw, _ps = np.meshgrid(np.arange(_J), np.arange(_NW), np.arange(_BLK), indexing="ij")
_P_FULL = _PERM.reshape(_NW, _BPW)[_pw, _ps // 4] * _S + 4 * _pj + (_ps % 4)
_P_FULL = _P_FULL.reshape(-1).astype(np.int32)


@jax.jit
def kernel(input_ids, tables):
    ids_shuf = (
        input_ids.reshape(_N)[jnp.asarray(_P_FULL)]
        .reshape(_J, _NW, 4, 128)
    )
    table_flat = _format_table(tables)
    out_sc = _sc_kernel(ids_shuf, table_flat)
    out2d = _format_out(out_sc)
    return out2d.reshape(_J, _D, _B).transpose(2, 0, 1)
